# Initial kernel scaffold; baseline (speedup 1.0000x reference)
import jax
import jax.numpy as jnp
from jax.experimental import pallas as pl


def kernel(x, W_enc, b_enc, W_dec, b_dec):
    raise NotImplementedError("write your pallas kernel here")


# speedup vs baseline: 10.5111x; 10.5111x over previous
import jax
import jax.numpy as jnp
from jax.experimental import pallas as pl
from jax.experimental.pallas import tpu as pltpu

B = 16384
D = 128
L = 1024
K = 163
CHUNK = 128
GRID = L // CHUNK


def _body(x_ref, we_ref, be_ref, wd_ref, bd_ref, dec_ref, sparse_ref):
    j = pl.program_id(0)

    x = x_ref[...]
    we = we_ref[...]
    enc = jax.lax.dot_general(
        x, we, (((1,), (1,)), ((), ())),
        preferred_element_type=jnp.float32)
    enc = enc + be_ref[...]

    bits = jax.lax.bitcast_convert_type(enc, jnp.int32)
    m = jnp.where(bits < 0, bits ^ jnp.int32(0x7FFFFFFF), bits)

    need = jnp.full((1, CHUNK), float(K), jnp.float32)
    cnt = jnp.sum((m >= 0).astype(jnp.float32), axis=0, keepdims=True)
    take = cnt >= need
    prefix = jnp.where(take, jnp.int32(0), jnp.int32(-1))
    need = jnp.where(take, need, need - cnt)
    for b in range(30, -1, -1):
        cand = prefix * 2 + 1
        eq = (m >> b) == cand
        cnt = jnp.sum(eq.astype(jnp.float32), axis=0, keepdims=True)
        take = cnt >= need
        prefix = jnp.where(take, cand, cand - 1)
        need = jnp.where(take, need, need - cnt)

    sp = jnp.where(m >= prefix, enc, 0.0)
    sparse_ref[...] = sp

    part = jax.lax.dot_general(
        sp, wd_ref[...], (((1,), (1,)), ((), ())),
        preferred_element_type=jnp.float32)

    @pl.when(j == 0)
    def _():
        dec_ref[...] = part + bd_ref[...]

    @pl.when(j > 0)
    def _():
        dec_ref[...] = dec_ref[...] + part


@jax.jit
def kernel(x, W_enc, b_enc, W_dec, b_dec):
    decoded, sparse = pl.pallas_call(
        _body,
        grid=(GRID,),
        in_specs=[
            pl.BlockSpec((B, D), lambda j: (0, 0)),
            pl.BlockSpec((CHUNK, D), lambda j: (j, 0)),
            pl.BlockSpec((1, CHUNK), lambda j: (0, j)),
            pl.BlockSpec((D, CHUNK), lambda j: (0, j)),
            pl.BlockSpec((1, D), lambda j: (0, 0)),
        ],
        out_specs=[
            pl.BlockSpec((B, D), lambda j: (0, 0)),
            pl.BlockSpec((B, CHUNK), lambda j: (0, j)),
        ],
        out_shape=[
            jax.ShapeDtypeStruct((B, D), jnp.float32),
            jax.ShapeDtypeStruct((B, L), jnp.float32),
        ],
    )(x, W_enc, b_enc.reshape(1, L), W_dec, b_dec.reshape(1, D))
    return (decoded, sparse)


# speedup vs baseline: 33.1569x; 3.1545x over previous
import jax
import jax.numpy as jnp
from jax.experimental import pallas as pl
from jax.experimental.pallas import tpu as pltpu

B = 16384
D = 128
L = 1024
K = 163
CHUNK = 128
GRID = L // CHUNK


def _body(x_ref, we_ref, be_ref, wd_ref, bd_ref, dec_ref, sparse_ref):
    j = pl.program_id(0)

    x = x_ref[...]
    we = we_ref[...]
    enc = jax.lax.dot_general(
        x, we, (((1,), (1,)), ((), ())),
        preferred_element_type=jnp.float32)
    enc = enc + be_ref[...]

    bits = jax.lax.bitcast_convert_type(enc, jnp.int32)
    m = jnp.where(bits < 0, bits ^ jnp.int32(0x7FFFFFFF), bits)

    ones = jnp.ones((1, B), jnp.bfloat16)

    def count_ge(t):
        ge = (m >= t).astype(jnp.bfloat16)
        return jax.lax.dot_general(
            ones, ge, (((1,), (0,)), ((), ())),
            preferred_element_type=jnp.float32)

    kf = jnp.float32(K)
    t = jnp.where(count_ge(jnp.int32(0)) >= kf,
                  jnp.int32(0), jnp.iinfo(jnp.int32).min)
    for b in range(30, -1, -1):
        t_hi = t + jnp.int32(1 << b)
        t = jnp.where(count_ge(t_hi) >= kf, t_hi, t)

    sp = jnp.where(m >= t, enc, 0.0)
    sparse_ref[...] = sp

    part = jax.lax.dot_general(
        sp, wd_ref[...], (((1,), (1,)), ((), ())),
        preferred_element_type=jnp.float32)

    @pl.when(j == 0)
    def _():
        dec_ref[...] = part + bd_ref[...]

    @pl.when(j > 0)
    def _():
        dec_ref[...] = dec_ref[...] + part


@jax.jit
def kernel(x, W_enc, b_enc, W_dec, b_dec):
    decoded, sparse = pl.pallas_call(
        _body,
        grid=(GRID,),
        in_specs=[
            pl.BlockSpec((B, D), lambda j: (0, 0)),
            pl.BlockSpec((CHUNK, D), lambda j: (j, 0)),
            pl.BlockSpec((1, CHUNK), lambda j: (0, j)),
            pl.BlockSpec((D, CHUNK), lambda j: (0, j)),
            pl.BlockSpec((1, D), lambda j: (0, 0)),
        ],
        out_specs=[
            pl.BlockSpec((B, D), lambda j: (0, 0)),
            pl.BlockSpec((B, CHUNK), lambda j: (0, j)),
        ],
        out_shape=[
            jax.ShapeDtypeStruct((B, D), jnp.float32),
            jax.ShapeDtypeStruct((B, L), jnp.float32),
        ],
    )(x, W_enc, b_enc.reshape(1, L), W_dec, b_dec.reshape(1, D))
    return (decoded, sparse)
